# Initial kernel scaffold; baseline (speedup 1.0000x reference)
import jax
import jax.numpy as jnp
from jax.experimental import pallas as pl


def kernel(input, table):
    raise NotImplementedError("write your pallas kernel here")


# speedup vs baseline: 1.5473x; 1.5473x over previous
import functools

import jax
import jax.numpy as jnp
from jax import lax
from jax.experimental import pallas as pl
from jax.experimental.pallas import tpu as pltpu
from jax.experimental.pallas import tpu_sc as plsc

EMBEDDING_DIM = 32
NUM_CORES = 2
NUM_SUBCORES = 16
NUM_WORKERS = NUM_CORES * NUM_SUBCORES
CHUNK = 1024


def _make_gather(num_idx: int):
  per_w = num_idx // NUM_WORKERS
  n_chunks = per_w // CHUNK
  assert per_w % CHUNK == 0 and num_idx % NUM_WORKERS == 0

  mesh = plsc.VectorSubcoreMesh(
      core_axis_name="c", subcore_axis_name="s",
      num_cores=NUM_CORES, num_subcores=NUM_SUBCORES)

  @functools.partial(
      pl.kernel,
      mesh=mesh,
      compiler_params=pltpu.CompilerParams(use_tc_tiling_on_sc=False),
      out_type=jax.ShapeDtypeStruct((num_idx, EMBEDDING_DIM), jnp.float32),
      scratch_types=[
          pltpu.VMEM((CHUNK,), jnp.int32),
          pltpu.VMEM((CHUNK, EMBEDDING_DIM), jnp.float32),
          pltpu.SemaphoreType.DMA,
      ],
  )
  def gather_kernel(idx_hbm, tab_hbm, out_hbm, idx_v, rows_v, sem):
    wid = lax.axis_index("s") * NUM_CORES + lax.axis_index("c")
    base = wid * per_w

    def step(g, carry):
      off = base + g * CHUNK
      pltpu.sync_copy(idx_hbm.at[pl.ds(off, CHUNK)], idx_v)
      pltpu.async_copy(tab_hbm.at[idx_v], rows_v, sem).wait()
      pltpu.sync_copy(rows_v, out_hbm.at[pl.ds(off, CHUNK)])
      return carry

    lax.fori_loop(0, n_chunks, step, 0, unroll=False)

  return gather_kernel


def kernel(input, table):
  batch, fields = input.shape
  num_idx = batch * fields
  idx = input.reshape(num_idx).astype(jnp.int32)
  out = _make_gather(num_idx)(idx, table)
  return out.reshape(batch, fields, EMBEDDING_DIM)


# speedup vs baseline: 1.5668x; 1.0126x over previous
import functools

import jax
import jax.numpy as jnp
from jax import lax
from jax.experimental import pallas as pl
from jax.experimental.pallas import tpu as pltpu
from jax.experimental.pallas import tpu_sc as plsc

EMBEDDING_DIM = 32
NUM_CORES = 2
NUM_SUBCORES = 16
NUM_WORKERS = NUM_CORES * NUM_SUBCORES
CHUNK = 1024


def _make_gather(num_idx: int):
  per_w = num_idx // NUM_WORKERS
  n_chunks = per_w // CHUNK
  assert per_w % CHUNK == 0 and num_idx % NUM_WORKERS == 0

  mesh = plsc.VectorSubcoreMesh(
      core_axis_name="c", subcore_axis_name="s",
      num_cores=NUM_CORES, num_subcores=NUM_SUBCORES)

  @functools.partial(
      pl.kernel,
      mesh=mesh,
      compiler_params=pltpu.CompilerParams(use_tc_tiling_on_sc=False),
      out_type=jax.ShapeDtypeStruct((num_idx, EMBEDDING_DIM), jnp.float32),
      scratch_types=[
          pltpu.VMEM((2, CHUNK), jnp.int32),
          pltpu.VMEM((2, CHUNK, EMBEDDING_DIM), jnp.float32),
          pltpu.SemaphoreType.DMA,
          pltpu.SemaphoreType.DMA,
      ],
  )
  def gather_kernel(idx_hbm, tab_hbm, out_hbm, idx_v, rows_v, sem_g, sem_o):
    wid = lax.axis_index("s") * NUM_CORES + lax.axis_index("c")
    base = wid * per_w

    gathers = [None] * n_chunks
    stores = [None] * n_chunks
    for g in range(n_chunks):
      s = g % 2
      off = base + g * CHUNK
      if g >= 2:
        stores[g - 2].wait()
      pltpu.sync_copy(idx_hbm.at[pl.ds(off, CHUNK)], idx_v.at[s])
      gathers[g] = pltpu.async_copy(tab_hbm.at[idx_v.at[s]], rows_v.at[s],
                                    sem_g)
      if g >= 1:
        gathers[g - 1].wait()
        stores[g - 1] = pltpu.async_copy(
            rows_v.at[1 - s], out_hbm.at[pl.ds(off - CHUNK, CHUNK)], sem_o)
    last = n_chunks - 1
    gathers[last].wait()
    stores[last] = pltpu.async_copy(
        rows_v.at[last % 2], out_hbm.at[pl.ds(base + last * CHUNK, CHUNK)],
        sem_o)
    stores[last - 1].wait()
    stores[last].wait()

  return gather_kernel


def kernel(input, table):
  batch, fields = input.shape
  num_idx = batch * fields
  idx = input.reshape(num_idx).astype(jnp.int32)
  out = _make_gather(num_idx)(idx, table)
  return out.reshape(batch, fields, EMBEDDING_DIM)
